# baseline (speedup 1.0000x reference)
import functools

import numpy as np
import jax
import jax.numpy as jnp
from jax import lax
from jax.experimental import pallas as pl
from jax.experimental.pallas import tpu as pltpu
from jax.experimental.pallas import tpu_sc as plsc

_FEAT_STRIDE = 16
_SCALES = np.array([8.0, 16.0, 32.0])
_RATIOS = np.array([0.5, 1.0, 2.0])
_PRE_NMS = 6000
_POST_NMS = 300
_NMS_THRESH = 0.7
_A = 9
_INT_MIN = np.int32(-(2 ** 31))
_NC = 6144
_TRASH = 6100
_NQ = 4
_NW = 32


def _whctrs(a):
    w = a[2] - a[0] + 1.0
    h = a[3] - a[1] + 1.0
    xc = a[0] + 0.5 * (w - 1.0)
    yc = a[1] + 0.5 * (h - 1.0)
    return w, h, xc, yc


def _mkanchors(ws, hs, xc, yc):
    ws = ws[:, None]
    hs = hs[:, None]
    return np.hstack([xc - 0.5 * (ws - 1.0), yc - 0.5 * (hs - 1.0),
                      xc + 0.5 * (ws - 1.0), yc + 0.5 * (hs - 1.0)])


def _ratio_enum(a, ratios):
    w, h, xc, yc = _whctrs(a)
    size = w * h
    size_ratios = size / ratios
    ws = np.round(np.sqrt(size_ratios))
    hs = np.round(ws * ratios)
    return _mkanchors(ws, hs, xc, yc)


def _scale_enum(a, scales):
    w, h, xc, yc = _whctrs(a)
    ws = w * scales
    hs = h * scales
    return _mkanchors(ws, hs, xc, yc)


def _gen_anchors(base_size=16):
    base = np.array([0.0, 0.0, base_size - 1.0, base_size - 1.0])
    ra = _ratio_enum(base, _RATIOS)
    return np.vstack([_scale_enum(ra[i, :], _SCALES) for i in range(ra.shape[0])])


def _anchor_geom_amajor(fh, fw, npad):
    anc = _gen_anchors().astype(np.float32)
    sx = np.arange(fw, dtype=np.float32) * np.float32(_FEAT_STRIDE)
    sy = np.arange(fh, dtype=np.float32) * np.float32(_FEAT_STRIDE)
    SX, SY = np.meshgrid(sx, sy)
    shifts = np.stack([SX.ravel(), SY.ravel(), SX.ravel(), SY.ravel()], axis=1).astype(np.float32)
    a4 = (anc[:, None, :] + shifts[None, :, :]).reshape(_A * fh * fw, 4)
    w = (a4[:, 2] - a4[:, 0]) + np.float32(1.0)
    h = (a4[:, 3] - a4[:, 1]) + np.float32(1.0)
    cx = a4[:, 0] + np.float32(0.5) * w
    cy = a4[:, 1] + np.float32(0.5) * h
    hw = fh * fw
    ridx = (np.arange(_A * hw, dtype=np.int64) % hw) * _A + (np.arange(_A * hw, dtype=np.int64) // hw)
    ridx = ridx.astype(np.int32)
    n = _A * hw

    def pad(v, c):
        return np.concatenate([v, np.full((npad - n,), c, v.dtype)])

    return (pad(w, 1.0), pad(h, 1.0), pad(cx, 0.0), pad(cy, 0.0),
            pad(ridx, np.int32(10 ** 8)))


def _stage_a(dx_r, dy_r, dw_r, dh_r, sc_r, aw_r, ah_r, acx_r, acy_r, ridx_r,
             imi_r, x1_o, y1_o, x2_o, y2_o, ar_o, ws_o, gsx_o):
    nb, n = sc_r.shape
    widths = aw_r[...]
    heights = ah_r[...]
    pcx = dx_r[...] * widths + acx_r[...]
    pcy = dy_r[...] * heights + acy_r[...]
    pw = jnp.exp(dw_r[...]) * widths
    ph = jnp.exp(dh_r[...]) * heights
    x1 = pcx - 0.5 * pw
    y1 = pcy - 0.5 * ph
    x2 = pcx + 0.5 * pw
    y2 = pcy + 0.5 * ph
    im_h = imi_r[:, 0:1]
    im_w = imi_r[:, 1:2]
    x1 = jnp.clip(x1, 0.0, im_w - 1.0)
    x2 = jnp.clip(x2, 0.0, im_w - 1.0)
    y1 = jnp.clip(y1, 0.0, im_h - 1.0)
    y2 = jnp.clip(y2, 0.0, im_h - 1.0)
    x1_o[...] = x1
    y1_o[...] = y1
    x2_o[...] = x2
    y2_o[...] = y2
    ar_o[...] = ((x2 - x1) + 1.0) * ((y2 - y1) + 1.0)

    sc = sc_r[...]
    bits = lax.bitcast_convert_type(sc, jnp.int32)
    key = jnp.where(bits >= 0, bits, _INT_MIN - bits)

    def bs_body(t, T):
        cand = T + (jnp.int32(1) << (jnp.int32(31) - t))
        cnt = jnp.sum((key >= cand).astype(jnp.int32), axis=1, keepdims=True)
        return jnp.where(cnt >= _PRE_NMS, cand, T)

    T = lax.fori_loop(0, 32, bs_body, jnp.full((nb, 1), _INT_MIN, jnp.int32))
    c_gt = jnp.sum((key > T).astype(jnp.int32), axis=1, keepdims=True)
    m = _PRE_NMS - c_gt
    ridx = ridx_r[...]
    tie = key == T

    def is_body(t, I):
        cand = I + (jnp.int32(1) << (jnp.int32(16) - t))
        f = jnp.sum((tie & (ridx < cand)).astype(jnp.int32), axis=1, keepdims=True)
        return jnp.where(f < m, cand, I)

    I = lax.fori_loop(0, 17, is_body, jnp.zeros((nb, 1), jnp.int32))
    part = (key > T) | (tie & (ridx <= I))
    ws_o[...] = jnp.where(part, sc, jnp.float32(-jnp.inf))

    p32 = part.astype(jnp.int32)
    s = p32
    sh = 1
    while sh < n:
        s = s + jnp.concatenate([jnp.zeros((nb, sh), jnp.int32), s[:, :n - sh]], axis=1)
        sh *= 2
    rank = s - p32
    bvec = lax.broadcasted_iota(jnp.int32, (nb, n), 0) * _NC
    gsx_o[...] = bvec + jnp.where(part, rank, jnp.int32(_TRASH))


def _sc_compact_body(gsx_h, x1_h, y1_h, x2_h, y2_h, ar_h, ws_h, ridx_h,
                     x1_o, y1_o, x2_o, y2_o, ar_o, ws_o, ridx_o,
                     gsx_v, x1_v, y1_v, x2_v, y2_v, ar_v, ws_v, ridx_v, sem):
    wid = lax.axis_index("s") * 2 + lax.axis_index("c")
    b = wid // _NQ
    q = wid % _NQ
    pltpu.sync_copy(gsx_h.at[b, q], gsx_v)
    pltpu.sync_copy(x1_h.at[b, q], x1_v)
    pltpu.sync_copy(y1_h.at[b, q], y1_v)
    pltpu.sync_copy(x2_h.at[b, q], x2_v)
    pltpu.sync_copy(y2_h.at[b, q], y2_v)
    pltpu.sync_copy(ar_h.at[b, q], ar_v)
    pltpu.sync_copy(ws_h.at[b, q], ws_v)
    pltpu.sync_copy(ridx_h.at[0, q], ridx_v)
    nrow = gsx_v.shape[0]
    pairs = ((x1_v, x1_o), (y1_v, y1_o), (x2_v, x2_o), (y2_v, y2_o),
             (ar_v, ar_o), (ws_v, ws_o), (ridx_v, ridx_o))

    def fire(j, c):
        for v, o in pairs:
            pltpu.make_async_copy(v.at[j], o.at[gsx_v.at[j]], sem).start()
        return c

    lax.fori_loop(0, nrow, fire, 0)

    def drain(j, c):
        for v, o in pairs:
            pltpu.make_async_copy(v.at[j], o.at[gsx_v.at[j]], sem).wait()
        return c

    lax.fori_loop(0, nrow, drain, 0)


def _stage_b(x1_r, y1_r, x2_r, y2_r, ar_r, ws_in, ridx_in, out_r, ws_r, ridx_r):
    nb, n = ws_in.shape
    pos = lax.broadcasted_iota(jnp.int32, (nb, n), 1)
    valid = pos < _PRE_NMS
    ninf = jnp.float32(-jnp.inf)
    ws_r[...] = jnp.where(valid, ws_in[...], ninf)
    ridx_r[...] = jnp.where(valid, ridx_in[...], jnp.int32(10 ** 9))
    lane = lax.broadcasted_iota(jnp.int32, (nb, 128), 1)
    z = jnp.zeros((nb, 1), jnp.float32)

    def step(i, fb):
        f1, f2, f3, f4 = fb
        ws = ws_r[...]
        ridx = ridx_r[...]
        mx = jnp.max(ws, axis=1, keepdims=True)
        selc = jnp.where(ws == mx, ridx, jnp.int32(10 ** 9))
        sel = jnp.min(selc, axis=1, keepdims=True)
        oh = ridx == sel
        bx1 = jnp.sum(jnp.where(oh, x1_r[...], 0.0), axis=1, keepdims=True)
        by1 = jnp.sum(jnp.where(oh, y1_r[...], 0.0), axis=1, keepdims=True)
        bx2 = jnp.sum(jnp.where(oh, x2_r[...], 0.0), axis=1, keepdims=True)
        by2 = jnp.sum(jnp.where(oh, y2_r[...], 0.0), axis=1, keepdims=True)
        bar = ((bx2 - bx1) + 1.0) * ((by2 - by1) + 1.0)
        alive = mx > ninf
        ox1 = jnp.where(alive, bx1, f1)
        oy1 = jnp.where(alive, by1, f2)
        ox2 = jnp.where(alive, bx2, f3)
        oy2 = jnp.where(alive, by2, f4)
        isz = i == 0
        f1 = jnp.where(isz, bx1, f1)
        f2 = jnp.where(isz, by1, f2)
        f3 = jnp.where(isz, bx2, f3)
        f4 = jnp.where(isz, by2, f4)
        xx1 = jnp.maximum(bx1, x1_r[...])
        yy1 = jnp.maximum(by1, y1_r[...])
        xx2 = jnp.minimum(bx2, x2_r[...])
        yy2 = jnp.minimum(by2, y2_r[...])
        w = jnp.maximum(0.0, (xx2 - xx1) + 1.0)
        h = jnp.maximum(0.0, (yy2 - yy1) + 1.0)
        inter = w * h
        iou = inter / ((bar + ar_r[...]) - inter)
        ws_r[...] = jnp.where(iou > _NMS_THRESH, ninf, ws)
        tile = jnp.where(lane == 0, ox1,
                         jnp.where(lane == 1, oy1,
                                   jnp.where(lane == 2, ox2,
                                             jnp.where(lane == 3, oy2, 0.0))))
        out_r[i] = tile
        return (f1, f2, f3, f4)

    lax.fori_loop(0, _POST_NMS, step, (z, z, z, z))


def _compact(batch, npad, span, gsx, x1, y1, x2, y2, ar, ws, ridx):
    f32 = jnp.float32
    r4 = lambda v: v.reshape(batch, _NQ, span // 128, 128)
    mesh = plsc.VectorSubcoreMesh(core_axis_name="c", subcore_axis_name="s")
    vm = lambda dt: pltpu.VMEM((span // 128, 128), dt)
    sc_fn = functools.partial(
        pl.kernel,
        mesh=mesh,
        out_type=[jax.ShapeDtypeStruct((batch * _NC,), f32)] * 6
        + [jax.ShapeDtypeStruct((batch * _NC,), jnp.int32)],
        scratch_types=[vm(jnp.int32)] + [vm(f32)] * 6 + [vm(jnp.int32),
                                                         pltpu.SemaphoreType.DMA],
    )(_sc_compact_body)
    return sc_fn(
        r4(gsx), r4(x1), r4(y1), r4(x2), r4(y2), r4(ar), r4(ws),
        ridx.reshape(1, _NQ, span // 128, 128))


def kernel(scores, bbox_deltas, im_info, cfg_key):
    batch = scores.shape[0]
    fh, fw = scores.shape[2], scores.shape[3]
    n = _A * fh * fw
    npad = ((n + 127) // 128) * 128
    padn = npad - n
    span = npad // _NQ

    sc = scores[:, _A:, :, :].reshape(batch, n)
    dx = bbox_deltas[:, 0::4, :, :].reshape(batch, n)
    dy = bbox_deltas[:, 1::4, :, :].reshape(batch, n)
    dw = bbox_deltas[:, 2::4, :, :].reshape(batch, n)
    dh = bbox_deltas[:, 3::4, :, :].reshape(batch, n)
    sc = jnp.pad(sc, ((0, 0), (0, padn)), constant_values=-jnp.inf)
    dx, dy, dw, dh = (jnp.pad(v, ((0, 0), (0, padn))) for v in (dx, dy, dw, dh))
    aw, ah, acx, acy, ridx = _anchor_geom_amajor(fh, fw, npad)
    aw, ah, acx, acy, ridx = (jnp.asarray(v)[None, :] for v in (aw, ah, acx, acy, ridx))
    imi = jnp.pad(im_info, ((0, 0), (0, 126)))

    f32 = jnp.float32
    x1, y1, x2, y2, ar, ws, gsx = pl.pallas_call(
        _stage_a,
        out_shape=[jax.ShapeDtypeStruct((batch, npad), f32)] * 6
        + [jax.ShapeDtypeStruct((batch, npad), jnp.int32)],
    )(dx, dy, dw, dh, sc, aw, ah, acx, acy, ridx, imi)

    cx1, cy1, cx2, cy2, car, cws, cridx = _compact(
        batch, npad, span, gsx, x1, y1, x2, y2, ar, ws, ridx)

    r2 = lambda v: v.reshape(batch, _NC)
    out = pl.pallas_call(
        _stage_b,
        out_shape=jax.ShapeDtypeStruct((_POST_NMS, batch, 128), f32),
        scratch_shapes=[pltpu.VMEM((batch, _NC), f32),
                        pltpu.VMEM((batch, _NC), jnp.int32)],
    )(r2(cx1), r2(cy1), r2(cx2), r2(cy2), r2(car), r2(cws), r2(cridx))

    kept = out[:, :, 0:4].transpose(1, 0, 2)
    batch_ids = jnp.broadcast_to(
        jnp.arange(batch, dtype=jnp.float32)[:, None, None], (batch, _POST_NMS, 1))
    return jnp.concatenate([batch_ids, kept], axis=2)


# speedup vs baseline: 48.8067x; 48.8067x over previous
import functools

import numpy as np
import jax
import jax.numpy as jnp
from jax import lax
from jax.experimental import pallas as pl
from jax.experimental.pallas import tpu as pltpu
from jax.experimental.pallas import tpu_sc as plsc

_FEAT_STRIDE = 16
_SCALES = np.array([8.0, 16.0, 32.0])
_RATIOS = np.array([0.5, 1.0, 2.0])
_PRE_NMS = 6000
_POST_NMS = 300
_NMS_THRESH = 0.7
_A = 9
_INT_MIN = np.int32(-(2 ** 31))
_NC = 6144
_TRASH = 6100
_NQ = 4
_NW = 32


def _whctrs(a):
    w = a[2] - a[0] + 1.0
    h = a[3] - a[1] + 1.0
    xc = a[0] + 0.5 * (w - 1.0)
    yc = a[1] + 0.5 * (h - 1.0)
    return w, h, xc, yc


def _mkanchors(ws, hs, xc, yc):
    ws = ws[:, None]
    hs = hs[:, None]
    return np.hstack([xc - 0.5 * (ws - 1.0), yc - 0.5 * (hs - 1.0),
                      xc + 0.5 * (ws - 1.0), yc + 0.5 * (hs - 1.0)])


def _ratio_enum(a, ratios):
    w, h, xc, yc = _whctrs(a)
    size = w * h
    size_ratios = size / ratios
    ws = np.round(np.sqrt(size_ratios))
    hs = np.round(ws * ratios)
    return _mkanchors(ws, hs, xc, yc)


def _scale_enum(a, scales):
    w, h, xc, yc = _whctrs(a)
    ws = w * scales
    hs = h * scales
    return _mkanchors(ws, hs, xc, yc)


def _gen_anchors(base_size=16):
    base = np.array([0.0, 0.0, base_size - 1.0, base_size - 1.0])
    ra = _ratio_enum(base, _RATIOS)
    return np.vstack([_scale_enum(ra[i, :], _SCALES) for i in range(ra.shape[0])])


def _anchor_geom_amajor(fh, fw, npad):
    anc = _gen_anchors().astype(np.float32)
    sx = np.arange(fw, dtype=np.float32) * np.float32(_FEAT_STRIDE)
    sy = np.arange(fh, dtype=np.float32) * np.float32(_FEAT_STRIDE)
    SX, SY = np.meshgrid(sx, sy)
    shifts = np.stack([SX.ravel(), SY.ravel(), SX.ravel(), SY.ravel()], axis=1).astype(np.float32)
    a4 = (anc[:, None, :] + shifts[None, :, :]).reshape(_A * fh * fw, 4)
    w = (a4[:, 2] - a4[:, 0]) + np.float32(1.0)
    h = (a4[:, 3] - a4[:, 1]) + np.float32(1.0)
    cx = a4[:, 0] + np.float32(0.5) * w
    cy = a4[:, 1] + np.float32(0.5) * h
    hw = fh * fw
    ridx = (np.arange(_A * hw, dtype=np.int64) % hw) * _A + (np.arange(_A * hw, dtype=np.int64) // hw)
    ridx = ridx.astype(np.int32)
    n = _A * hw

    def pad(v, c):
        return np.concatenate([v, np.full((npad - n,), c, v.dtype)])

    return (pad(w, 1.0), pad(h, 1.0), pad(cx, 0.0), pad(cy, 0.0),
            pad(ridx, np.int32(10 ** 8)))


def _stage_a(dx_r, dy_r, dw_r, dh_r, sc_r, aw_r, ah_r, acx_r, acy_r, ridx_r,
             imi_r, p_o, gsx_o):
    nb, n = sc_r.shape
    widths = aw_r[...]
    heights = ah_r[...]
    pcx = dx_r[...] * widths + acx_r[...]
    pcy = dy_r[...] * heights + acy_r[...]
    pw = jnp.exp(dw_r[...]) * widths
    ph = jnp.exp(dh_r[...]) * heights
    x1 = pcx - 0.5 * pw
    y1 = pcy - 0.5 * ph
    x2 = pcx + 0.5 * pw
    y2 = pcy + 0.5 * ph
    im_h = imi_r[:, 0:1]
    im_w = imi_r[:, 1:2]
    x1 = jnp.clip(x1, 0.0, im_w - 1.0)
    x2 = jnp.clip(x2, 0.0, im_w - 1.0)
    y1 = jnp.clip(y1, 0.0, im_h - 1.0)
    y2 = jnp.clip(y2, 0.0, im_h - 1.0)
    p_o[0] = x1
    p_o[1] = y1
    p_o[2] = x2
    p_o[3] = y2
    p_o[4] = ((x2 - x1) + 1.0) * ((y2 - y1) + 1.0)

    sc = sc_r[...]
    bits = lax.bitcast_convert_type(sc, jnp.int32)
    key = jnp.where(bits >= 0, bits, _INT_MIN - bits)

    def bs_body(t, T):
        cand = T + (jnp.int32(1) << (jnp.int32(31) - t))
        cnt = jnp.sum((key >= cand).astype(jnp.int32), axis=1, keepdims=True)
        return jnp.where(cnt >= _PRE_NMS, cand, T)

    T = lax.fori_loop(0, 32, bs_body, jnp.full((nb, 1), _INT_MIN, jnp.int32))
    c_gt = jnp.sum((key > T).astype(jnp.int32), axis=1, keepdims=True)
    m = _PRE_NMS - c_gt
    ridx = ridx_r[...]
    tie = key == T

    def is_body(t, I):
        cand = I + (jnp.int32(1) << (jnp.int32(16) - t))
        f = jnp.sum((tie & (ridx < cand)).astype(jnp.int32), axis=1, keepdims=True)
        return jnp.where(f < m, cand, I)

    I = lax.fori_loop(0, 17, is_body, jnp.zeros((nb, 1), jnp.int32))
    part = (key > T) | (tie & (ridx <= I))
    p_o[5] = jnp.where(part, sc, jnp.float32(-jnp.inf))
    ridxf = lax.bitcast_convert_type(jnp.broadcast_to(ridx, (nb, n)), jnp.float32)
    p_o[6] = ridxf
    p_o[7] = ridxf

    p32 = part.astype(jnp.int32)
    s = p32
    sh = 1
    while sh < n:
        s = s + jnp.concatenate([jnp.zeros((nb, sh), jnp.int32), s[:, :n - sh]], axis=1)
        sh *= 2
    rank = s - p32
    gsx_o[...] = jnp.where(part, rank, jnp.int32(_TRASH))


def _sc_compact_body(p_h, gsx_h, out_h, a_v, b_v, gsx_v, outa_v, outb_v):
    wid = lax.axis_index("s") * 2 + lax.axis_index("c")
    b = wid // _NQ
    q = wid % _NQ
    p0 = q * 2
    p1 = p0 + 1
    pltpu.sync_copy(p_h.at[p0, b], a_v)
    pltpu.sync_copy(p_h.at[p1, b], b_v)
    pltpu.sync_copy(gsx_h.at[b], gsx_v)
    n = gsx_v.shape[0]

    def body(j, c):
        sl = pl.ds(j * 16, 16)
        idx = gsx_v[sl]
        plsc.store_scatter(outa_v, [idx], a_v[sl])
        plsc.store_scatter(outb_v, [idx], b_v[sl])
        return c

    lax.fori_loop(0, n // 16, body, 0)
    pltpu.sync_copy(outa_v, out_h.at[p0, b])
    pltpu.sync_copy(outb_v, out_h.at[p1, b])


def _stage_b(p_r, out_r, ws_r, ridx_r):
    _, nb, n = p_r.shape
    x1_v = p_r[0]
    y1_v = p_r[1]
    x2_v = p_r[2]
    y2_v = p_r[3]
    ar_v = p_r[4]
    pos = lax.broadcasted_iota(jnp.int32, (nb, n), 1)
    valid = pos < _PRE_NMS
    ninf = jnp.float32(-jnp.inf)
    ws_r[...] = jnp.where(valid, p_r[5], ninf)
    ridx_r[...] = jnp.where(valid, lax.bitcast_convert_type(p_r[6], jnp.int32),
                            jnp.int32(10 ** 9))
    lane = lax.broadcasted_iota(jnp.int32, (nb, 128), 1)
    z = jnp.zeros((nb, 1), jnp.float32)

    def step(i, fb):
        f1, f2, f3, f4 = fb
        ws = ws_r[...]
        ridx = ridx_r[...]
        mx = jnp.max(ws, axis=1, keepdims=True)
        selc = jnp.where(ws == mx, ridx, jnp.int32(10 ** 9))
        sel = jnp.min(selc, axis=1, keepdims=True)
        oh = ridx == sel
        bx1 = jnp.sum(jnp.where(oh, x1_v, 0.0), axis=1, keepdims=True)
        by1 = jnp.sum(jnp.where(oh, y1_v, 0.0), axis=1, keepdims=True)
        bx2 = jnp.sum(jnp.where(oh, x2_v, 0.0), axis=1, keepdims=True)
        by2 = jnp.sum(jnp.where(oh, y2_v, 0.0), axis=1, keepdims=True)
        bar = ((bx2 - bx1) + 1.0) * ((by2 - by1) + 1.0)
        alive = mx > ninf
        ox1 = jnp.where(alive, bx1, f1)
        oy1 = jnp.where(alive, by1, f2)
        ox2 = jnp.where(alive, bx2, f3)
        oy2 = jnp.where(alive, by2, f4)
        isz = i == 0
        f1 = jnp.where(isz, bx1, f1)
        f2 = jnp.where(isz, by1, f2)
        f3 = jnp.where(isz, bx2, f3)
        f4 = jnp.where(isz, by2, f4)
        xx1 = jnp.maximum(bx1, x1_v)
        yy1 = jnp.maximum(by1, y1_v)
        xx2 = jnp.minimum(bx2, x2_v)
        yy2 = jnp.minimum(by2, y2_v)
        w = jnp.maximum(0.0, (xx2 - xx1) + 1.0)
        h = jnp.maximum(0.0, (yy2 - yy1) + 1.0)
        inter = w * h
        iou = inter / ((bar + ar_v) - inter)
        ws_r[...] = jnp.where(iou > _NMS_THRESH, ninf, ws)
        tile = jnp.where(lane == 0, ox1,
                         jnp.where(lane == 1, oy1,
                                   jnp.where(lane == 2, ox2,
                                             jnp.where(lane == 3, oy2, 0.0))))
        out_r[i] = tile
        return (f1, f2, f3, f4)

    lax.fori_loop(0, _POST_NMS, step, (z, z, z, z))


def _compact(batch, npad, planes, gsx):
    f32 = jnp.float32
    mesh = plsc.VectorSubcoreMesh(core_axis_name="c", subcore_axis_name="s")
    sc_fn = functools.partial(
        pl.kernel,
        mesh=mesh,
        compiler_params=pltpu.CompilerParams(needs_layout_passes=False),
        out_type=jax.ShapeDtypeStruct((2 * _NQ, batch, _NC), f32),
        scratch_types=[pltpu.VMEM((npad,), f32), pltpu.VMEM((npad,), f32),
                       pltpu.VMEM((npad,), jnp.int32),
                       pltpu.VMEM((_NC,), f32), pltpu.VMEM((_NC,), f32)],
    )(_sc_compact_body)
    return sc_fn(planes, gsx)


def kernel(scores, bbox_deltas, im_info, cfg_key):
    batch = scores.shape[0]
    fh, fw = scores.shape[2], scores.shape[3]
    n = _A * fh * fw
    npad = ((n + 127) // 128) * 128
    padn = npad - n
    span = npad // _NQ

    sc = scores[:, _A:, :, :].reshape(batch, n)
    dx = bbox_deltas[:, 0::4, :, :].reshape(batch, n)
    dy = bbox_deltas[:, 1::4, :, :].reshape(batch, n)
    dw = bbox_deltas[:, 2::4, :, :].reshape(batch, n)
    dh = bbox_deltas[:, 3::4, :, :].reshape(batch, n)
    sc = jnp.pad(sc, ((0, 0), (0, padn)), constant_values=-jnp.inf)
    dx, dy, dw, dh = (jnp.pad(v, ((0, 0), (0, padn))) for v in (dx, dy, dw, dh))
    aw, ah, acx, acy, ridx = _anchor_geom_amajor(fh, fw, npad)
    aw, ah, acx, acy, ridx = (jnp.asarray(v)[None, :] for v in (aw, ah, acx, acy, ridx))
    imi = jnp.pad(im_info, ((0, 0), (0, 126)))

    f32 = jnp.float32
    planes, gsx = pl.pallas_call(
        _stage_a,
        out_shape=[jax.ShapeDtypeStruct((2 * _NQ, batch, npad), f32),
                   jax.ShapeDtypeStruct((batch, npad), jnp.int32)],
    )(dx, dy, dw, dh, sc, aw, ah, acx, acy, ridx, imi)

    cplanes = _compact(batch, npad, planes, gsx)

    out = pl.pallas_call(
        _stage_b,
        out_shape=jax.ShapeDtypeStruct((_POST_NMS, batch, 128), f32),
        scratch_shapes=[pltpu.VMEM((batch, _NC), f32),
                        pltpu.VMEM((batch, _NC), jnp.int32)],
    )(cplanes)

    kept = out[:, :, 0:4].transpose(1, 0, 2)
    batch_ids = jnp.broadcast_to(
        jnp.arange(batch, dtype=jnp.float32)[:, None, None], (batch, _POST_NMS, 1))
    return jnp.concatenate([batch_ids, kept], axis=2)


# speedup vs baseline: 49.1257x; 1.0065x over previous
import functools

import numpy as np
import jax
import jax.numpy as jnp
from jax import lax
from jax.experimental import pallas as pl
from jax.experimental.pallas import tpu as pltpu
from jax.experimental.pallas import tpu_sc as plsc

_FEAT_STRIDE = 16
_SCALES = np.array([8.0, 16.0, 32.0])
_RATIOS = np.array([0.5, 1.0, 2.0])
_PRE_NMS = 6000
_POST_NMS = 300
_NMS_THRESH = 0.7
_A = 9
_INT_MIN = np.int32(-(2 ** 31))
_NC = 6144
_TRASH = 6100
_NQ = 4


def _whctrs(a):
    w = a[2] - a[0] + 1.0
    h = a[3] - a[1] + 1.0
    xc = a[0] + 0.5 * (w - 1.0)
    yc = a[1] + 0.5 * (h - 1.0)
    return w, h, xc, yc


def _mkanchors(ws, hs, xc, yc):
    ws = ws[:, None]
    hs = hs[:, None]
    return np.hstack([xc - 0.5 * (ws - 1.0), yc - 0.5 * (hs - 1.0),
                      xc + 0.5 * (ws - 1.0), yc + 0.5 * (hs - 1.0)])


def _ratio_enum(a, ratios):
    w, h, xc, yc = _whctrs(a)
    size = w * h
    size_ratios = size / ratios
    ws = np.round(np.sqrt(size_ratios))
    hs = np.round(ws * ratios)
    return _mkanchors(ws, hs, xc, yc)


def _scale_enum(a, scales):
    w, h, xc, yc = _whctrs(a)
    ws = w * scales
    hs = h * scales
    return _mkanchors(ws, hs, xc, yc)


def _gen_anchors(base_size=16):
    base = np.array([0.0, 0.0, base_size - 1.0, base_size - 1.0])
    ra = _ratio_enum(base, _RATIOS)
    return np.vstack([_scale_enum(ra[i, :], _SCALES) for i in range(ra.shape[0])])


def _anchor_geom_amajor(fh, fw, npad):
    anc = _gen_anchors().astype(np.float32)
    sx = np.arange(fw, dtype=np.float32) * np.float32(_FEAT_STRIDE)
    sy = np.arange(fh, dtype=np.float32) * np.float32(_FEAT_STRIDE)
    SX, SY = np.meshgrid(sx, sy)
    shifts = np.stack([SX.ravel(), SY.ravel(), SX.ravel(), SY.ravel()], axis=1).astype(np.float32)
    a4 = (anc[:, None, :] + shifts[None, :, :]).reshape(_A * fh * fw, 4)
    w = (a4[:, 2] - a4[:, 0]) + np.float32(1.0)
    h = (a4[:, 3] - a4[:, 1]) + np.float32(1.0)
    cx = a4[:, 0] + np.float32(0.5) * w
    cy = a4[:, 1] + np.float32(0.5) * h
    hw = fh * fw
    ridx = (np.arange(_A * hw, dtype=np.int64) % hw) * _A + (np.arange(_A * hw, dtype=np.int64) // hw)
    ridx = ridx.astype(np.int32)
    n = _A * hw

    def pad(v, c):
        return np.concatenate([v, np.full((npad - n,), c, v.dtype)])

    return (pad(w, 1.0), pad(h, 1.0), pad(cx, 0.0), pad(cy, 0.0),
            pad(ridx, np.int32(10 ** 8)))


def _stage_a(dx_r, dy_r, dw_r, dh_r, sc_r, aw_r, ah_r, acx_r, acy_r, ridx_r,
             imi_r, p_o, gsx_o):
    nb, n = sc_r.shape
    widths = aw_r[...]
    heights = ah_r[...]
    pcx = dx_r[...] * widths + acx_r[...]
    pcy = dy_r[...] * heights + acy_r[...]
    pw = jnp.exp(dw_r[...]) * widths
    ph = jnp.exp(dh_r[...]) * heights
    x1 = pcx - 0.5 * pw
    y1 = pcy - 0.5 * ph
    x2 = pcx + 0.5 * pw
    y2 = pcy + 0.5 * ph
    im_h = imi_r[:, 0:1]
    im_w = imi_r[:, 1:2]
    x1 = jnp.clip(x1, 0.0, im_w - 1.0)
    x2 = jnp.clip(x2, 0.0, im_w - 1.0)
    y1 = jnp.clip(y1, 0.0, im_h - 1.0)
    y2 = jnp.clip(y2, 0.0, im_h - 1.0)
    p_o[0] = x1
    p_o[1] = y1
    p_o[2] = x2
    p_o[3] = y2
    p_o[4] = ((x2 - x1) + 1.0) * ((y2 - y1) + 1.0)

    sc = sc_r[...]
    bits = lax.bitcast_convert_type(sc, jnp.int32)
    key = jnp.where(bits >= 0, bits, _INT_MIN - bits)

    def bs_body(t, T):
        cand = T + (jnp.int32(1) << (jnp.int32(31) - t))
        cnt = jnp.sum((key >= cand).astype(jnp.int32), axis=1, keepdims=True)
        return jnp.where(cnt >= _PRE_NMS, cand, T)

    T = lax.fori_loop(0, 32, bs_body, jnp.full((nb, 1), _INT_MIN, jnp.int32))
    c_gt = jnp.sum((key > T).astype(jnp.int32), axis=1, keepdims=True)
    m = _PRE_NMS - c_gt
    ridx = ridx_r[...]
    tie = key == T

    def is_body(t, I):
        cand = I + (jnp.int32(1) << (jnp.int32(16) - t))
        f = jnp.sum((tie & (ridx < cand)).astype(jnp.int32), axis=1, keepdims=True)
        return jnp.where(f < m, cand, I)

    I = lax.fori_loop(0, 17, is_body, jnp.zeros((nb, 1), jnp.int32))
    part = (key > T) | (tie & (ridx <= I))
    p_o[5] = jnp.where(part, sc, jnp.float32(-jnp.inf))
    ridxf = lax.bitcast_convert_type(jnp.broadcast_to(ridx, (nb, n)), jnp.float32)
    p_o[6] = ridxf
    p_o[7] = ridxf

    p32 = part.astype(jnp.int32)
    s = p32
    sh = 1
    while sh < n:
        s = s + jnp.concatenate([jnp.zeros((nb, sh), jnp.int32), s[:, :n - sh]], axis=1)
        sh *= 2
    rank = s - p32
    gsx_o[...] = jnp.where(part, rank, jnp.int32(_TRASH))


def _sc_compact_body(p_h, gsx_h, out_h, a_v, b_v, gsx_v, outa_v, outb_v):
    wid = lax.axis_index("s") * 2 + lax.axis_index("c")
    b = wid // _NQ
    q = wid % _NQ
    p0 = q * 2
    p1 = p0 + 1
    pltpu.sync_copy(p_h.at[p0, b], a_v)
    pltpu.sync_copy(p_h.at[p1, b], b_v)
    pltpu.sync_copy(gsx_h.at[b], gsx_v)
    n = gsx_v.shape[0]

    def body(j, c):
        sl = pl.ds(j * 16, 16)
        idx = gsx_v[sl]
        plsc.store_scatter(outa_v, [idx], a_v[sl])
        plsc.store_scatter(outb_v, [idx], b_v[sl])
        return c

    lax.fori_loop(0, n // 16, body, 0)
    pltpu.sync_copy(outa_v, out_h.at[p0, b])
    pltpu.sync_copy(outb_v, out_h.at[p1, b])


def _stage_b(p_r, out_r, ws_r, ridx_r):
    _, nb, n = p_r.shape
    x1_v = p_r[0]
    y1_v = p_r[1]
    x2_v = p_r[2]
    y2_v = p_r[3]
    ar_v = p_r[4]
    pos = lax.broadcasted_iota(jnp.int32, (nb, n), 1)
    valid = pos < _PRE_NMS
    ninf = jnp.float32(-jnp.inf)
    ws_r[...] = jnp.where(valid, p_r[5], ninf)
    ridx_r[...] = jnp.where(valid, lax.bitcast_convert_type(p_r[6], jnp.int32),
                            jnp.int32(10 ** 9))
    lane = lax.broadcasted_iota(jnp.int32, (nb, 128), 1)
    z = jnp.zeros((nb, 1), jnp.float32)

    def step(i, fb):
        f1, f2, f3, f4 = fb
        ws = ws_r[...]
        ridx = ridx_r[...]
        mx = jnp.max(ws, axis=1, keepdims=True)
        selc = jnp.where(ws == mx, ridx, jnp.int32(10 ** 9))
        sel = jnp.min(selc, axis=1, keepdims=True)
        oh = ridx == sel
        bx1 = jnp.sum(jnp.where(oh, x1_v, 0.0), axis=1, keepdims=True)
        by1 = jnp.sum(jnp.where(oh, y1_v, 0.0), axis=1, keepdims=True)
        bx2 = jnp.sum(jnp.where(oh, x2_v, 0.0), axis=1, keepdims=True)
        by2 = jnp.sum(jnp.where(oh, y2_v, 0.0), axis=1, keepdims=True)
        bar = ((bx2 - bx1) + 1.0) * ((by2 - by1) + 1.0)
        alive = mx > ninf
        ox1 = jnp.where(alive, bx1, f1)
        oy1 = jnp.where(alive, by1, f2)
        ox2 = jnp.where(alive, bx2, f3)
        oy2 = jnp.where(alive, by2, f4)
        isz = i == 0
        f1 = jnp.where(isz, bx1, f1)
        f2 = jnp.where(isz, by1, f2)
        f3 = jnp.where(isz, bx2, f3)
        f4 = jnp.where(isz, by2, f4)
        xx1 = jnp.maximum(bx1, x1_v)
        yy1 = jnp.maximum(by1, y1_v)
        xx2 = jnp.minimum(bx2, x2_v)
        yy2 = jnp.minimum(by2, y2_v)
        w = jnp.maximum(0.0, (xx2 - xx1) + 1.0)
        h = jnp.maximum(0.0, (yy2 - yy1) + 1.0)
        inter = w * h
        iou = inter / ((bar + ar_v) - inter)
        ws_r[...] = jnp.where(iou > _NMS_THRESH, ninf, ws)
        tile = jnp.where(lane == 0, ox1,
                         jnp.where(lane == 1, oy1,
                                   jnp.where(lane == 2, ox2,
                                             jnp.where(lane == 3, oy2, 0.0))))
        out_r[i] = tile
        return (f1, f2, f3, f4)

    lax.fori_loop(0, _POST_NMS, step, (z, z, z, z))


def _compact(batch, npad, planes, gsx):
    f32 = jnp.float32
    mesh = plsc.VectorSubcoreMesh(core_axis_name="c", subcore_axis_name="s")
    sc_fn = functools.partial(
        pl.kernel,
        mesh=mesh,
        compiler_params=pltpu.CompilerParams(needs_layout_passes=False),
        out_type=jax.ShapeDtypeStruct((2 * _NQ, batch, _NC), f32),
        scratch_types=[pltpu.VMEM((npad,), f32), pltpu.VMEM((npad,), f32),
                       pltpu.VMEM((npad,), jnp.int32),
                       pltpu.VMEM((_NC,), f32), pltpu.VMEM((_NC,), f32)],
    )(_sc_compact_body)
    return sc_fn(planes, gsx)


def kernel(scores, bbox_deltas, im_info, cfg_key):
    batch = scores.shape[0]
    fh, fw = scores.shape[2], scores.shape[3]
    n = _A * fh * fw
    npad = ((n + 127) // 128) * 128
    padn = npad - n

    sc = scores[:, _A:, :, :].reshape(batch, n)
    dx = bbox_deltas[:, 0::4, :, :].reshape(batch, n)
    dy = bbox_deltas[:, 1::4, :, :].reshape(batch, n)
    dw = bbox_deltas[:, 2::4, :, :].reshape(batch, n)
    dh = bbox_deltas[:, 3::4, :, :].reshape(batch, n)
    sc = jnp.pad(sc, ((0, 0), (0, padn)), constant_values=-jnp.inf)
    dx, dy, dw, dh = (jnp.pad(v, ((0, 0), (0, padn))) for v in (dx, dy, dw, dh))
    aw, ah, acx, acy, ridx = _anchor_geom_amajor(fh, fw, npad)
    aw, ah, acx, acy, ridx = (jnp.asarray(v)[None, :] for v in (aw, ah, acx, acy, ridx))
    imi = jnp.pad(im_info, ((0, 0), (0, 126)))

    f32 = jnp.float32
    planes, gsx = pl.pallas_call(
        _stage_a,
        out_shape=[jax.ShapeDtypeStruct((2 * _NQ, batch, npad), f32),
                   jax.ShapeDtypeStruct((batch, npad), jnp.int32)],
    )(dx, dy, dw, dh, sc, aw, ah, acx, acy, ridx, imi)

    cplanes = _compact(batch, npad, planes, gsx)

    out = pl.pallas_call(
        _stage_b,
        out_shape=jax.ShapeDtypeStruct((_POST_NMS, batch, 128), f32),
        scratch_shapes=[pltpu.VMEM((batch, _NC), f32),
                        pltpu.VMEM((batch, _NC), jnp.int32)],
    )(cplanes)

    kept = out[:, :, 0:4].transpose(1, 0, 2)
    batch_ids = jnp.broadcast_to(
        jnp.arange(batch, dtype=jnp.float32)[:, None, None], (batch, _POST_NMS, 1))
    return jnp.concatenate([batch_ids, kept], axis=2)
